# baseline (device time: 136776 ns/iter reference)
import jax
import jax.numpy as jnp
from jax import lax
from jax.experimental import pallas as pl
from jax.experimental.pallas import tpu as pltpu

T = 4096
T_HALF = T // 2
V_SHARD = 8192
D = 2048

CHUNK_SIZES = [32, 32, 64, 96] + [128] * 13 + [96, 64]
CHUNK_STARTS = [sum(CHUNK_SIZES[:i]) for i in range(len(CHUNK_SIZES))]
N_CHUNKS = len(CHUNK_SIZES)
G_MAX = max(CHUNK_SIZES)
assert sum(CHUNK_SIZES) == T_HALF
L1, L2 = 2, 4


def kernel(ids, E):
    def body(rc_ref, comp_ref, cnt_ref, mask_ref, e_ref, out_ref,
             gbuf, ysend, yrecv, xrecv,
             gsems, osem, ysend_sems, yrecv_sems, xsend_sems, xrecv_sems):
        my_x = lax.axis_index("x")
        my_y = lax.axis_index("y")

        barrier_sem = pltpu.get_barrier_semaphore()
        pl.semaphore_signal(barrier_sem, inc=1, device_id=(my_x, 1 - my_y),
                            device_id_type=pl.DeviceIdType.MESH)
        pl.semaphore_signal(barrier_sem, inc=1, device_id=(1 - my_x, my_y),
                            device_id_type=pl.DeviceIdType.MESH)
        pl.semaphore_wait(barrier_sem, 2)

        def row_copy(i, base, slot):
            return pltpu.make_async_copy(
                e_ref.at[pl.ds(rc_ref[i], 1)],
                gbuf.at[slot, pl.ds(i - base, 1)],
                gsems.at[slot],
            )

        def issue_block(c):
            base, slot = CHUNK_STARTS[c], c % 2

            def f(j, _):
                p = comp_ref[j]
                row_copy(p, base, slot).start()
                return 0

            lax.fori_loop(base, base + cnt_ref[c], f, 0)

        def drain_block(c):
            dummy = pltpu.make_async_copy(
                e_ref.at[pl.ds(0, 1)], gbuf.at[c % 2, pl.ds(0, 1)],
                gsems.at[c % 2],
            )

            def f(i, _):
                dummy.wait()
                return 0

            lax.fori_loop(0, cnt_ref[c], f, 0)

        def y_rdma(c):
            ch = pl.ds(CHUNK_STARTS[c], CHUNK_SIZES[c])
            return pltpu.make_async_remote_copy(
                src_ref=ysend.at[ch],
                dst_ref=yrecv.at[ch],
                send_sem=ysend_sems.at[c],
                recv_sem=yrecv_sems.at[c],
                device_id=(my_x, 1 - my_y),
                device_id_type=pl.DeviceIdType.MESH,
            )

        def x_rdma(c):
            ch = pl.ds(CHUNK_STARTS[c], CHUNK_SIZES[c])
            return pltpu.make_async_remote_copy(
                src_ref=ysend.at[ch],
                dst_ref=xrecv.at[ch],
                send_sem=xsend_sems.at[c],
                recv_sem=xrecv_sems.at[c],
                device_id=(1 - my_x, my_y),
                device_id_type=pl.DeviceIdType.MESH,
            )

        def own_store(c):
            ch = pl.ds(CHUNK_STARTS[c], CHUNK_SIZES[c])
            return pltpu.make_async_copy(
                ysend.at[ch],
                out_ref.at[pl.ds(my_x * T_HALF + CHUNK_STARTS[c],
                                 CHUNK_SIZES[c])],
                osem,
            )

        def other_store(c):
            ch = pl.ds(CHUNK_STARTS[c], CHUNK_SIZES[c])
            return pltpu.make_async_copy(
                xrecv.at[ch],
                out_ref.at[pl.ds((1 - my_x) * T_HALF + CHUNK_STARTS[c],
                                 CHUNK_SIZES[c])],
                osem,
            )

        def gather_and_ysend(c):
            drain_block(c)
            base, sz = CHUNK_STARTS[c], CHUNK_SIZES[c]
            ch = pl.ds(base, sz)
            ysend[ch] = jnp.where(
                mask_ref[ch] > 0, gbuf[c % 2, pl.ds(0, sz)], 0.0
            ).astype(jnp.bfloat16)
            y_rdma(c).start()

        def reduce_and_xsend(c):
            y_rdma(c).wait()
            ch = pl.ds(CHUNK_STARTS[c], CHUNK_SIZES[c])
            ysend[ch] = ysend[ch] + yrecv[ch]
            x_rdma(c).start()
            own_store(c).start()

        def store_other(c):
            x_rdma(c).wait()
            other_store(c).start()

        issue_block(0)
        for c in range(N_CHUNKS + L2):
            if c + 1 < N_CHUNKS:
                issue_block(c + 1)
            if c < N_CHUNKS:
                gather_and_ysend(c)
            if L1 <= c < N_CHUNKS + L1:
                reduce_and_xsend(c - L1)
            if L2 <= c:
                store_other(c - L2)

        for c in range(N_CHUNKS):
            own_store(c).wait()
            other_store(c).wait()

    my_x = lax.axis_index("x")
    my_y = lax.axis_index("y")
    ids_half = lax.dynamic_slice(ids, (my_x * T_HALF,), (T_HALF,))
    r = ids_half - my_y * V_SHARD
    in_range = (r >= 0) & (r < V_SHARD)
    mask = in_range.astype(jnp.float32).reshape(T_HALF, 1)
    rc = jnp.where(in_range, r, -1).astype(jnp.int32)
    cum = jnp.concatenate(
        [jnp.zeros((1,), jnp.int32), jnp.cumsum(in_range.astype(jnp.int32))]
    )
    ends = jnp.array([s + z for s, z in zip(CHUNK_STARTS, CHUNK_SIZES)])
    starts = jnp.array(CHUNK_STARTS)
    cnts = (cum[ends] - cum[starts]).astype(jnp.int32)
    chunk_of = jnp.array(
        sum(([i] * z for i, z in enumerate(CHUNK_SIZES)), []), dtype=jnp.int32
    )
    prefix = cum[:-1] - cum[starts][chunk_of]
    pos = jnp.arange(T_HALF, dtype=jnp.int32)
    slot_idx = jnp.where(in_range, starts[chunk_of] + prefix, T_HALF)
    comp = jnp.zeros((T_HALF,), jnp.int32).at[slot_idx].set(pos, mode="drop")

    return pl.pallas_call(
        body,
        out_shape=jax.ShapeDtypeStruct((T, D), jnp.bfloat16),
        in_specs=[
            pl.BlockSpec(memory_space=pltpu.SMEM),
            pl.BlockSpec(memory_space=pltpu.SMEM),
            pl.BlockSpec(memory_space=pltpu.SMEM),
            pl.BlockSpec(memory_space=pltpu.VMEM),
            pl.BlockSpec(memory_space=pl.ANY),
        ],
        out_specs=pl.BlockSpec(memory_space=pl.ANY),
        scratch_shapes=[
            pltpu.VMEM((2, G_MAX, D), jnp.float32),
            pltpu.VMEM((T_HALF, D), jnp.bfloat16),
            pltpu.VMEM((T_HALF, D), jnp.bfloat16),
            pltpu.VMEM((T_HALF, D), jnp.bfloat16),
            pltpu.SemaphoreType.DMA((2,)),
            pltpu.SemaphoreType.DMA,
            pltpu.SemaphoreType.DMA((N_CHUNKS,)),
            pltpu.SemaphoreType.DMA((N_CHUNKS,)),
            pltpu.SemaphoreType.DMA((N_CHUNKS,)),
            pltpu.SemaphoreType.DMA((N_CHUNKS,)),
        ],
        compiler_params=pltpu.CompilerParams(collective_id=0),
    )(rc, comp, cnts, mask, E)


# device time: 121720 ns/iter; 1.1237x vs baseline; 1.1237x over previous
import jax
import jax.numpy as jnp
from jax import lax
from jax.experimental import pallas as pl
from jax.experimental.pallas import tpu as pltpu

T = 4096
T_HALF = T // 2
V_SHARD = 8192
D = 2048

CHUNK_SIZES = [32, 32, 64, 96, 128] + [256] * 6 + [96, 64]
CHUNK_STARTS = [sum(CHUNK_SIZES[:i]) for i in range(len(CHUNK_SIZES))]
N_CHUNKS = len(CHUNK_SIZES)
G_MAX = max(CHUNK_SIZES)
assert sum(CHUNK_SIZES) == T_HALF
L1, L2 = 2, 4


def kernel(ids, E):
    def body(rc_ref, cnt_ref, mask_ref, e_ref, out_ref,
             gbuf, ysend, yrecv, xrecv,
             gsems, osem, ysend_sems, yrecv_sems, xsend_sems, xrecv_sems):
        my_x = lax.axis_index("x")
        my_y = lax.axis_index("y")

        barrier_sem = pltpu.get_barrier_semaphore()
        pl.semaphore_signal(barrier_sem, inc=1, device_id=(my_x, 1 - my_y),
                            device_id_type=pl.DeviceIdType.MESH)
        pl.semaphore_signal(barrier_sem, inc=1, device_id=(1 - my_x, my_y),
                            device_id_type=pl.DeviceIdType.MESH)
        pl.semaphore_wait(barrier_sem, 2)

        def row_copy(i, base, slot):
            return pltpu.make_async_copy(
                e_ref.at[pl.ds(rc_ref[i], 1)],
                gbuf.at[slot, pl.ds(i - base, 1)],
                gsems.at[slot],
            )

        def issue_block(c):
            base, slot = CHUNK_STARTS[c], c % 2

            def f(i, _):
                @pl.when(rc_ref[i] >= 0)
                def _():
                    row_copy(i, base, slot).start()

                return 0

            lax.fori_loop(base, base + CHUNK_SIZES[c], f, 0, unroll=4)

        def drain_block(c):
            dummy = pltpu.make_async_copy(
                e_ref.at[pl.ds(0, 1)], gbuf.at[c % 2, pl.ds(0, 1)],
                gsems.at[c % 2],
            )

            def f(i, _):
                dummy.wait()
                return 0

            lax.fori_loop(0, cnt_ref[c], f, 0)

        def y_rdma(c):
            ch = pl.ds(CHUNK_STARTS[c], CHUNK_SIZES[c])
            return pltpu.make_async_remote_copy(
                src_ref=ysend.at[ch],
                dst_ref=yrecv.at[ch],
                send_sem=ysend_sems.at[c],
                recv_sem=yrecv_sems.at[c],
                device_id=(my_x, 1 - my_y),
                device_id_type=pl.DeviceIdType.MESH,
            )

        def x_rdma(c):
            ch = pl.ds(CHUNK_STARTS[c], CHUNK_SIZES[c])
            return pltpu.make_async_remote_copy(
                src_ref=ysend.at[ch],
                dst_ref=xrecv.at[ch],
                send_sem=xsend_sems.at[c],
                recv_sem=xrecv_sems.at[c],
                device_id=(1 - my_x, my_y),
                device_id_type=pl.DeviceIdType.MESH,
            )

        def own_store(c):
            ch = pl.ds(CHUNK_STARTS[c], CHUNK_SIZES[c])
            return pltpu.make_async_copy(
                ysend.at[ch],
                out_ref.at[pl.ds(my_x * T_HALF + CHUNK_STARTS[c],
                                 CHUNK_SIZES[c])],
                osem,
            )

        def other_store(c):
            ch = pl.ds(CHUNK_STARTS[c], CHUNK_SIZES[c])
            return pltpu.make_async_copy(
                xrecv.at[ch],
                out_ref.at[pl.ds((1 - my_x) * T_HALF + CHUNK_STARTS[c],
                                 CHUNK_SIZES[c])],
                osem,
            )

        def gather_and_ysend(c):
            drain_block(c)
            base, sz = CHUNK_STARTS[c], CHUNK_SIZES[c]
            ch = pl.ds(base, sz)
            ysend[ch] = jnp.where(
                mask_ref[ch] > 0, gbuf[c % 2, pl.ds(0, sz)], 0.0
            ).astype(jnp.bfloat16)
            y_rdma(c).start()

        def reduce_and_xsend(c):
            y_rdma(c).wait()
            ch = pl.ds(CHUNK_STARTS[c], CHUNK_SIZES[c])
            ysend[ch] = ysend[ch] + yrecv[ch]
            x_rdma(c).start()
            own_store(c).start()

        def store_other(c):
            x_rdma(c).wait()
            other_store(c).start()

        issue_block(0)
        for c in range(N_CHUNKS + L2):
            if c + 1 < N_CHUNKS:
                issue_block(c + 1)
            if c < N_CHUNKS:
                gather_and_ysend(c)
            if L1 <= c < N_CHUNKS + L1:
                reduce_and_xsend(c - L1)
            if L2 <= c:
                store_other(c - L2)

        for c in range(N_CHUNKS):
            own_store(c).wait()
            other_store(c).wait()

    my_x = lax.axis_index("x")
    my_y = lax.axis_index("y")
    ids_half = lax.dynamic_slice(ids, (my_x * T_HALF,), (T_HALF,))
    r = ids_half - my_y * V_SHARD
    in_range = (r >= 0) & (r < V_SHARD)
    mask = in_range.astype(jnp.float32).reshape(T_HALF, 1)
    rc = jnp.where(in_range, r, -1).astype(jnp.int32)
    cum = jnp.concatenate(
        [jnp.zeros((1,), jnp.int32), jnp.cumsum(in_range.astype(jnp.int32))]
    )
    ends = jnp.array([s + z for s, z in zip(CHUNK_STARTS, CHUNK_SIZES)])
    starts = jnp.array(CHUNK_STARTS)
    cnts = (cum[ends] - cum[starts]).astype(jnp.int32)

    return pl.pallas_call(
        body,
        out_shape=jax.ShapeDtypeStruct((T, D), jnp.bfloat16),
        in_specs=[
            pl.BlockSpec(memory_space=pltpu.SMEM),
            pl.BlockSpec(memory_space=pltpu.SMEM),
            pl.BlockSpec(memory_space=pltpu.VMEM),
            pl.BlockSpec(memory_space=pl.ANY),
        ],
        out_specs=pl.BlockSpec(memory_space=pl.ANY),
        scratch_shapes=[
            pltpu.VMEM((2, G_MAX, D), jnp.float32),
            pltpu.VMEM((T_HALF, D), jnp.bfloat16),
            pltpu.VMEM((T_HALF, D), jnp.bfloat16),
            pltpu.VMEM((T_HALF, D), jnp.bfloat16),
            pltpu.SemaphoreType.DMA((2,)),
            pltpu.SemaphoreType.DMA,
            pltpu.SemaphoreType.DMA((N_CHUNKS,)),
            pltpu.SemaphoreType.DMA((N_CHUNKS,)),
            pltpu.SemaphoreType.DMA((N_CHUNKS,)),
            pltpu.SemaphoreType.DMA((N_CHUNKS,)),
        ],
        compiler_params=pltpu.CompilerParams(collective_id=0),
    )(rc, cnts, mask, E)


# device time: 117718 ns/iter; 1.1619x vs baseline; 1.0340x over previous
import jax
import jax.numpy as jnp
from jax import lax
from jax.experimental import pallas as pl
from jax.experimental.pallas import tpu as pltpu

T = 4096
T_HALF = T // 2
V_SHARD = 8192
D = 2048

CHUNK_SIZES = [32, 32, 64, 96] + [128] * 13 + [96, 64]
CHUNK_STARTS = [sum(CHUNK_SIZES[:i]) for i in range(len(CHUNK_SIZES))]
N_CHUNKS = len(CHUNK_SIZES)
G_MAX = max(CHUNK_SIZES)
assert sum(CHUNK_SIZES) == T_HALF
L1, L2 = 2, 4


def kernel(ids, E):
    def body(rc_ref, mask_ref, e_ref, out_ref,
             gbuf, ysend, yrecv, xrecv,
             gsems, osem, ysend_sems, yrecv_sems, xsend_sems, xrecv_sems):
        my_x = lax.axis_index("x")
        my_y = lax.axis_index("y")

        barrier_sem = pltpu.get_barrier_semaphore()
        pl.semaphore_signal(barrier_sem, inc=1, device_id=(my_x, 1 - my_y),
                            device_id_type=pl.DeviceIdType.MESH)
        pl.semaphore_signal(barrier_sem, inc=1, device_id=(1 - my_x, my_y),
                            device_id_type=pl.DeviceIdType.MESH)
        pl.semaphore_wait(barrier_sem, 2)

        def row_copy(i, base, slot):
            return pltpu.make_async_copy(
                e_ref.at[pl.ds(rc_ref[i], 1)],
                gbuf.at[slot, pl.ds(i - base, 1)],
                gsems.at[slot],
            )

        def issue_block(c):
            base, slot = CHUNK_STARTS[c], c % 2

            def f(i, _):
                row_copy(i, base, slot).start()
                return 0

            lax.fori_loop(base, base + CHUNK_SIZES[c], f, 0, unroll=4)

        def drain_block(c):
            dummy = pltpu.make_async_copy(
                e_ref.at[pl.ds(0, 1)], gbuf.at[c % 2, pl.ds(0, 1)],
                gsems.at[c % 2],
            )

            def f(i, _):
                dummy.wait()
                return 0

            lax.fori_loop(0, CHUNK_SIZES[c], f, 0, unroll=4)

        def y_rdma(c):
            ch = pl.ds(CHUNK_STARTS[c], CHUNK_SIZES[c])
            return pltpu.make_async_remote_copy(
                src_ref=ysend.at[ch],
                dst_ref=yrecv.at[ch],
                send_sem=ysend_sems.at[c],
                recv_sem=yrecv_sems.at[c],
                device_id=(my_x, 1 - my_y),
                device_id_type=pl.DeviceIdType.MESH,
            )

        def x_rdma(c):
            ch = pl.ds(CHUNK_STARTS[c], CHUNK_SIZES[c])
            return pltpu.make_async_remote_copy(
                src_ref=ysend.at[ch],
                dst_ref=xrecv.at[ch],
                send_sem=xsend_sems.at[c],
                recv_sem=xrecv_sems.at[c],
                device_id=(1 - my_x, my_y),
                device_id_type=pl.DeviceIdType.MESH,
            )

        def own_store(c):
            ch = pl.ds(CHUNK_STARTS[c], CHUNK_SIZES[c])
            return pltpu.make_async_copy(
                ysend.at[ch],
                out_ref.at[pl.ds(my_x * T_HALF + CHUNK_STARTS[c],
                                 CHUNK_SIZES[c])],
                osem,
            )

        def other_store(c):
            ch = pl.ds(CHUNK_STARTS[c], CHUNK_SIZES[c])
            return pltpu.make_async_copy(
                xrecv.at[ch],
                out_ref.at[pl.ds((1 - my_x) * T_HALF + CHUNK_STARTS[c],
                                 CHUNK_SIZES[c])],
                osem,
            )

        def gather_and_ysend(c):
            drain_block(c)
            base, sz = CHUNK_STARTS[c], CHUNK_SIZES[c]
            ch = pl.ds(base, sz)
            ysend[ch] = jnp.where(
                mask_ref[ch] > 0, gbuf[c % 2, pl.ds(0, sz)], 0.0
            ).astype(jnp.bfloat16)
            y_rdma(c).start()

        def reduce_and_xsend(c):
            y_rdma(c).wait()
            ch = pl.ds(CHUNK_STARTS[c], CHUNK_SIZES[c])
            ysend[ch] = ysend[ch] + yrecv[ch]
            x_rdma(c).start()
            own_store(c).start()

        def store_other(c):
            x_rdma(c).wait()
            other_store(c).start()

        issue_block(0)
        for c in range(N_CHUNKS + L2):
            if c + 1 < N_CHUNKS:
                issue_block(c + 1)
            if c < N_CHUNKS:
                gather_and_ysend(c)
            if L1 <= c < N_CHUNKS + L1:
                reduce_and_xsend(c - L1)
            if L2 <= c:
                store_other(c - L2)

        for c in range(N_CHUNKS):
            own_store(c).wait()
            other_store(c).wait()

    my_x = lax.axis_index("x")
    my_y = lax.axis_index("y")
    ids_half = lax.dynamic_slice(ids, (my_x * T_HALF,), (T_HALF,))
    r = ids_half - my_y * V_SHARD
    in_range = (r >= 0) & (r < V_SHARD)
    mask = in_range.astype(jnp.float32).reshape(T_HALF, 1)
    rc = jnp.clip(r, 0, V_SHARD - 1).astype(jnp.int32)

    return pl.pallas_call(
        body,
        out_shape=jax.ShapeDtypeStruct((T, D), jnp.bfloat16),
        in_specs=[
            pl.BlockSpec(memory_space=pltpu.SMEM),
            pl.BlockSpec(memory_space=pltpu.VMEM),
            pl.BlockSpec(memory_space=pl.ANY),
        ],
        out_specs=pl.BlockSpec(memory_space=pl.ANY),
        scratch_shapes=[
            pltpu.VMEM((2, G_MAX, D), jnp.float32),
            pltpu.VMEM((T_HALF, D), jnp.bfloat16),
            pltpu.VMEM((T_HALF, D), jnp.bfloat16),
            pltpu.VMEM((T_HALF, D), jnp.bfloat16),
            pltpu.SemaphoreType.DMA((2,)),
            pltpu.SemaphoreType.DMA,
            pltpu.SemaphoreType.DMA((N_CHUNKS,)),
            pltpu.SemaphoreType.DMA((N_CHUNKS,)),
            pltpu.SemaphoreType.DMA((N_CHUNKS,)),
            pltpu.SemaphoreType.DMA((N_CHUNKS,)),
        ],
        compiler_params=pltpu.CompilerParams(collective_id=0),
    )(rc, mask, E)


# device time: 117166 ns/iter; 1.1674x vs baseline; 1.0047x over previous
import jax
import jax.numpy as jnp
from jax import lax
from jax.experimental import pallas as pl
from jax.experimental.pallas import tpu as pltpu

T = 4096
T_HALF = T // 2
V_SHARD = 8192
D = 2048

CHUNK_SIZES = [32, 32, 64, 96] + [128] * 13 + [96, 64]
CHUNK_STARTS = [sum(CHUNK_SIZES[:i]) for i in range(len(CHUNK_SIZES))]
N_CHUNKS = len(CHUNK_SIZES)
G_MAX = max(CHUNK_SIZES)
assert sum(CHUNK_SIZES) == T_HALF
L1, L2 = 3, 5


def kernel(ids, E):
    def body(rc_ref, cnt_ref, mask_ref, e_ref, out_ref,
             gbuf, ysend, yrecv, xrecv,
             gsems, osem, ysend_sems, yrecv_sems, xsend_sems, xrecv_sems):
        my_x = lax.axis_index("x")
        my_y = lax.axis_index("y")

        barrier_sem = pltpu.get_barrier_semaphore()
        pl.semaphore_signal(barrier_sem, inc=1, device_id=(my_x, 1 - my_y),
                            device_id_type=pl.DeviceIdType.MESH)
        pl.semaphore_signal(barrier_sem, inc=1, device_id=(1 - my_x, my_y),
                            device_id_type=pl.DeviceIdType.MESH)
        pl.semaphore_wait(barrier_sem, 2)

        def row_copy(i, base, slot):
            return pltpu.make_async_copy(
                e_ref.at[pl.ds(rc_ref[i], 1)],
                gbuf.at[slot, pl.ds(i - base, 1)],
                gsems.at[slot],
            )

        def issue_block(c):
            base, slot = CHUNK_STARTS[c], c % 2

            def f(i, _):
                @pl.when(rc_ref[i] >= 0)
                def _():
                    row_copy(i, base, slot).start()

                return 0

            lax.fori_loop(base, base + CHUNK_SIZES[c], f, 0, unroll=4)

        def drain_block(c):
            dummy = pltpu.make_async_copy(
                e_ref.at[pl.ds(0, 1)], gbuf.at[c % 2, pl.ds(0, 1)],
                gsems.at[c % 2],
            )

            def f(i, _):
                dummy.wait()
                return 0

            lax.fori_loop(0, cnt_ref[c], f, 0)

        def y_rdma(c):
            ch = pl.ds(CHUNK_STARTS[c], CHUNK_SIZES[c])
            return pltpu.make_async_remote_copy(
                src_ref=ysend.at[ch],
                dst_ref=yrecv.at[ch],
                send_sem=ysend_sems.at[c],
                recv_sem=yrecv_sems.at[c],
                device_id=(my_x, 1 - my_y),
                device_id_type=pl.DeviceIdType.MESH,
            )

        def x_rdma(c):
            ch = pl.ds(CHUNK_STARTS[c], CHUNK_SIZES[c])
            return pltpu.make_async_remote_copy(
                src_ref=ysend.at[ch],
                dst_ref=xrecv.at[ch],
                send_sem=xsend_sems.at[c],
                recv_sem=xrecv_sems.at[c],
                device_id=(1 - my_x, my_y),
                device_id_type=pl.DeviceIdType.MESH,
            )

        def own_store(c):
            ch = pl.ds(CHUNK_STARTS[c], CHUNK_SIZES[c])
            return pltpu.make_async_copy(
                ysend.at[ch],
                out_ref.at[pl.ds(my_x * T_HALF + CHUNK_STARTS[c],
                                 CHUNK_SIZES[c])],
                osem,
            )

        def other_store(c):
            ch = pl.ds(CHUNK_STARTS[c], CHUNK_SIZES[c])
            return pltpu.make_async_copy(
                xrecv.at[ch],
                out_ref.at[pl.ds((1 - my_x) * T_HALF + CHUNK_STARTS[c],
                                 CHUNK_SIZES[c])],
                osem,
            )

        def gather_and_ysend(c):
            drain_block(c)
            base, sz = CHUNK_STARTS[c], CHUNK_SIZES[c]
            ch = pl.ds(base, sz)
            ysend[ch] = jnp.where(
                mask_ref[ch] > 0, gbuf[c % 2, pl.ds(0, sz)], 0.0
            ).astype(jnp.bfloat16)
            y_rdma(c).start()

        def reduce_and_xsend(c):
            y_rdma(c).wait()
            ch = pl.ds(CHUNK_STARTS[c], CHUNK_SIZES[c])
            ysend[ch] = ysend[ch] + yrecv[ch]
            x_rdma(c).start()
            own_store(c).start()

        def store_other(c):
            x_rdma(c).wait()
            other_store(c).start()

        issue_block(0)
        for c in range(N_CHUNKS + L2):
            if c + 1 < N_CHUNKS:
                issue_block(c + 1)
            if c < N_CHUNKS:
                gather_and_ysend(c)
            if L1 <= c < N_CHUNKS + L1:
                reduce_and_xsend(c - L1)
            if L2 <= c:
                store_other(c - L2)

        for c in range(N_CHUNKS):
            own_store(c).wait()
            other_store(c).wait()

    my_x = lax.axis_index("x")
    my_y = lax.axis_index("y")
    ids_half = lax.dynamic_slice(ids, (my_x * T_HALF,), (T_HALF,))
    r = ids_half - my_y * V_SHARD
    in_range = (r >= 0) & (r < V_SHARD)
    mask = in_range.astype(jnp.float32).reshape(T_HALF, 1)
    rc = jnp.where(in_range, r, -1).astype(jnp.int32)
    cum = jnp.concatenate(
        [jnp.zeros((1,), jnp.int32), jnp.cumsum(in_range.astype(jnp.int32))]
    )
    ends = jnp.array([s + z for s, z in zip(CHUNK_STARTS, CHUNK_SIZES)])
    cnts = (cum[ends] - cum[jnp.array(CHUNK_STARTS)]).astype(jnp.int32)

    return pl.pallas_call(
        body,
        out_shape=jax.ShapeDtypeStruct((T, D), jnp.bfloat16),
        in_specs=[
            pl.BlockSpec(memory_space=pltpu.SMEM),
            pl.BlockSpec(memory_space=pltpu.SMEM),
            pl.BlockSpec(memory_space=pltpu.VMEM),
            pl.BlockSpec(memory_space=pl.ANY),
        ],
        out_specs=pl.BlockSpec(memory_space=pl.ANY),
        scratch_shapes=[
            pltpu.VMEM((2, G_MAX, D), jnp.float32),
            pltpu.VMEM((T_HALF, D), jnp.bfloat16),
            pltpu.VMEM((T_HALF, D), jnp.bfloat16),
            pltpu.VMEM((T_HALF, D), jnp.bfloat16),
            pltpu.SemaphoreType.DMA((2,)),
            pltpu.SemaphoreType.DMA,
            pltpu.SemaphoreType.DMA((N_CHUNKS,)),
            pltpu.SemaphoreType.DMA((N_CHUNKS,)),
            pltpu.SemaphoreType.DMA((N_CHUNKS,)),
            pltpu.SemaphoreType.DMA((N_CHUNKS,)),
        ],
        compiler_params=pltpu.CompilerParams(collective_id=0),
    )(rc, cnts, mask, E)


# device time: 115243 ns/iter; 1.1868x vs baseline; 1.0167x over previous
import jax
import jax.numpy as jnp
from jax import lax
from jax.experimental import pallas as pl
from jax.experimental.pallas import tpu as pltpu

T = 4096
T_HALF = T // 2
V_SHARD = 8192
D = 2048

CHUNK_SIZES = [32, 32, 64, 96] + [128] * 13 + [96, 64]
CHUNK_STARTS = [sum(CHUNK_SIZES[:i]) for i in range(len(CHUNK_SIZES))]
N_CHUNKS = len(CHUNK_SIZES)
G_MAX = max(CHUNK_SIZES)
assert sum(CHUNK_SIZES) == T_HALF
L1, L2 = 2, 4


def kernel(ids, E):
    def body(rc_ref, cnt_ref, mask_ref, e_ref, out_ref,
             gbuf, ysend, yrecv, xrecv,
             gsems, osem, ysend_sems, yrecv_sems, xsend_sems, xrecv_sems):
        my_x = lax.axis_index("x")
        my_y = lax.axis_index("y")

        barrier_sem = pltpu.get_barrier_semaphore()
        pl.semaphore_signal(barrier_sem, inc=1, device_id=(my_x, 1 - my_y),
                            device_id_type=pl.DeviceIdType.MESH)
        pl.semaphore_signal(barrier_sem, inc=1, device_id=(1 - my_x, my_y),
                            device_id_type=pl.DeviceIdType.MESH)
        pl.semaphore_wait(barrier_sem, 2)

        def row_copy(i, base, slot):
            return pltpu.make_async_copy(
                e_ref.at[pl.ds(rc_ref[i], 1)],
                gbuf.at[slot, pl.ds(i - base, 1)],
                gsems.at[slot],
            )

        def issue_block(c):
            base, slot = CHUNK_STARTS[c], c % 2

            def f(i, _):
                @pl.when(rc_ref[i] >= 0)
                def _():
                    row_copy(i, base, slot).start()

                return 0

            lax.fori_loop(base, base + CHUNK_SIZES[c], f, 0, unroll=4)

        def drain_block(c):
            dummy = pltpu.make_async_copy(
                e_ref.at[pl.ds(0, 1)], gbuf.at[c % 2, pl.ds(0, 1)],
                gsems.at[c % 2],
            )

            def f(i, _):
                dummy.wait()
                return 0

            lax.fori_loop(0, cnt_ref[c], f, 0)

        def y_rdma(c):
            ch = pl.ds(CHUNK_STARTS[c], CHUNK_SIZES[c])
            return pltpu.make_async_remote_copy(
                src_ref=ysend.at[ch],
                dst_ref=yrecv.at[ch],
                send_sem=ysend_sems.at[c],
                recv_sem=yrecv_sems.at[c],
                device_id=(my_x, 1 - my_y),
                device_id_type=pl.DeviceIdType.MESH,
            )

        def x_rdma(c):
            ch = pl.ds(CHUNK_STARTS[c], CHUNK_SIZES[c])
            return pltpu.make_async_remote_copy(
                src_ref=ysend.at[ch],
                dst_ref=xrecv.at[ch],
                send_sem=xsend_sems.at[c],
                recv_sem=xrecv_sems.at[c],
                device_id=(1 - my_x, my_y),
                device_id_type=pl.DeviceIdType.MESH,
            )

        def own_store(c):
            ch = pl.ds(CHUNK_STARTS[c], CHUNK_SIZES[c])
            return pltpu.make_async_copy(
                ysend.at[ch],
                out_ref.at[pl.ds(my_x * T_HALF + CHUNK_STARTS[c],
                                 CHUNK_SIZES[c])],
                osem,
            )

        def other_store(c):
            ch = pl.ds(CHUNK_STARTS[c], CHUNK_SIZES[c])
            return pltpu.make_async_copy(
                xrecv.at[ch],
                out_ref.at[pl.ds((1 - my_x) * T_HALF + CHUNK_STARTS[c],
                                 CHUNK_SIZES[c])],
                osem,
            )

        def gather_and_ysend(c):
            drain_block(c)
            base, sz = CHUNK_STARTS[c], CHUNK_SIZES[c]
            ch = pl.ds(base, sz)
            ysend[ch] = jnp.where(
                mask_ref[ch] > 0, gbuf[c % 2, pl.ds(0, sz)], 0.0
            ).astype(jnp.bfloat16)
            y_rdma(c).start()

        def reduce_and_xsend(c):
            y_rdma(c).wait()
            ch = pl.ds(CHUNK_STARTS[c], CHUNK_SIZES[c])
            ysend[ch] = ysend[ch] + yrecv[ch]
            x_rdma(c).start()
            own_store(c).start()

        def store_other(c):
            x_rdma(c).wait()
            other_store(c).start()

        issue_block(0)
        for c in range(N_CHUNKS + L2):
            if c + 1 < N_CHUNKS:
                issue_block(c + 1)
            if c < N_CHUNKS:
                gather_and_ysend(c)
            if L1 <= c < N_CHUNKS + L1:
                reduce_and_xsend(c - L1)
            if L2 <= c:
                store_other(c - L2)

        for c in range(N_CHUNKS):
            own_store(c).wait()
            other_store(c).wait()

    my_x = lax.axis_index("x")
    my_y = lax.axis_index("y")
    ids_half = lax.dynamic_slice(ids, (my_x * T_HALF,), (T_HALF,))
    r = ids_half - my_y * V_SHARD
    in_range = (r >= 0) & (r < V_SHARD)
    mask = in_range.astype(jnp.float32).reshape(T_HALF, 1)
    rc = jnp.where(in_range, r, -1).astype(jnp.int32)
    cum = jnp.concatenate(
        [jnp.zeros((1,), jnp.int32), jnp.cumsum(in_range.astype(jnp.int32))]
    )
    ends = jnp.array([s + z for s, z in zip(CHUNK_STARTS, CHUNK_SIZES)])
    cnts = (cum[ends] - cum[jnp.array(CHUNK_STARTS)]).astype(jnp.int32)

    return pl.pallas_call(
        body,
        out_shape=jax.ShapeDtypeStruct((T, D), jnp.bfloat16),
        in_specs=[
            pl.BlockSpec(memory_space=pltpu.SMEM),
            pl.BlockSpec(memory_space=pltpu.SMEM),
            pl.BlockSpec(memory_space=pltpu.VMEM),
            pl.BlockSpec(memory_space=pl.ANY),
        ],
        out_specs=pl.BlockSpec(memory_space=pl.ANY),
        scratch_shapes=[
            pltpu.VMEM((2, G_MAX, D), jnp.float32),
            pltpu.VMEM((T_HALF, D), jnp.bfloat16),
            pltpu.VMEM((T_HALF, D), jnp.bfloat16),
            pltpu.VMEM((T_HALF, D), jnp.bfloat16),
            pltpu.SemaphoreType.DMA((2,)),
            pltpu.SemaphoreType.DMA,
            pltpu.SemaphoreType.DMA((N_CHUNKS,)),
            pltpu.SemaphoreType.DMA((N_CHUNKS,)),
            pltpu.SemaphoreType.DMA((N_CHUNKS,)),
            pltpu.SemaphoreType.DMA((N_CHUNKS,)),
        ],
        compiler_params=pltpu.CompilerParams(collective_id=0),
    )(rc, cnts, mask, E)
